# Initial kernel scaffold; baseline (speedup 1.0000x reference)
import jax
import jax.numpy as jnp
from jax.experimental import pallas as pl


def kernel(x, embeddings):
    raise NotImplementedError("write your pallas kernel here")


# speedup vs baseline: 1.8619x; 1.8619x over previous
import functools

import jax
import jax.numpy as jnp
from jax import lax
from jax.experimental import pallas as pl
from jax.experimental.pallas import tpu as pltpu
from jax.experimental.pallas import tpu_sc as plsc

NC = 2
NS = 16
NW = NC * NS


@functools.lru_cache(maxsize=None)
def _build(B: int, V: int, D: int, nbuf: int, C: int):
    assert B % NW == 0
    bpw = B // NW
    assert bpw % C == 0
    nchunk = bpw // C
    assert nchunk % nbuf == 0
    rounds = nchunk // nbuf

    mesh = plsc.VectorSubcoreMesh(
        core_axis_name="c", subcore_axis_name="s",
        num_cores=NC, num_subcores=NS)

    @functools.partial(
        pl.kernel,
        out_type=jax.ShapeDtypeStruct((B, D), jnp.float32),
        mesh=mesh,
        compiler_params=pltpu.CompilerParams(use_tc_tiling_on_sc=False),
        scratch_types=[
            [pltpu.VMEM((C,), jnp.int32)] * nbuf,
            [pltpu.VMEM((C, D), jnp.float32)] * nbuf,
            pltpu.SemaphoreType.DMA((nbuf,)),
            pltpu.SemaphoreType.DMA((nbuf,)),
        ],
    )
    def emb(idx_hbm, table_hbm, out_hbm, idx_v, rows_v, gsem, osem):
        wid = lax.axis_index("s") * NC + lax.axis_index("c")
        base = wid * bpw

        def start_gather(g, b):
            pltpu.sync_copy(idx_hbm.at[pl.ds(base + g * C, C)], idx_v[b])
            pltpu.async_copy(table_hbm.at[idx_v[b]], rows_v[b], gsem.at[b])

        def wait_gather(b):
            pltpu.make_async_copy(table_hbm.at[idx_v[b]], rows_v[b],
                                  gsem.at[b]).wait()

        def start_out(g, b):
            pltpu.async_copy(rows_v[b],
                             out_hbm.at[pl.ds(base + g * C, C)], osem.at[b])

        def wait_out(g, b):
            pltpu.make_async_copy(rows_v[b],
                                  out_hbm.at[pl.ds(base + g * C, C)],
                                  osem.at[b]).wait()

        for b in range(nbuf):
            start_gather(b, b)

        def round_body(i, carry):
            for b in range(nbuf):
                g = i * nbuf + b
                wait_gather(b)
                start_out(g, b)
                wait_out(g, b)

                @pl.when(i < rounds - 1)
                def _():
                    start_gather(g + nbuf, b)
            return carry

        lax.fori_loop(0, rounds, round_body, 0)

    return emb


def kernel(x, embeddings):
    bsz, hist = x.shape
    V, D = embeddings.shape
    B = bsz * hist
    xf = x.reshape(B).astype(jnp.int32)
    out = _build(B, V, D, 4, 400)(xf, embeddings)
    return out.reshape(bsz, hist, D)


# speedup vs baseline: 1.8740x; 1.0065x over previous
import functools

import jax
import jax.numpy as jnp
from jax import lax
from jax.experimental import pallas as pl
from jax.experimental.pallas import tpu as pltpu
from jax.experimental.pallas import tpu_sc as plsc

NC = 2
NS = 16
NW = NC * NS


@functools.lru_cache(maxsize=None)
def _build(B: int, V: int, D: int, nbuf: int, C: int):
    assert B % NW == 0
    bpw = B // NW
    assert bpw % C == 0
    nchunk = bpw // C
    assert nchunk % nbuf == 0
    rounds = nchunk // nbuf

    mesh = plsc.VectorSubcoreMesh(
        core_axis_name="c", subcore_axis_name="s",
        num_cores=NC, num_subcores=NS)

    @functools.partial(
        pl.kernel,
        out_type=jax.ShapeDtypeStruct((B, D), jnp.float32),
        mesh=mesh,
        compiler_params=pltpu.CompilerParams(use_tc_tiling_on_sc=False),
        scratch_types=[
            pltpu.VMEM((bpw,), jnp.int32),
            [pltpu.VMEM((C, D), jnp.float32)] * nbuf,
            pltpu.SemaphoreType.DMA((nbuf,)),
            pltpu.SemaphoreType.DMA((nbuf,)),
        ],
    )
    def emb(idx_hbm, table_hbm, out_hbm, idx_v, rows_v, gsem, osem):
        wid = lax.axis_index("s") * NC + lax.axis_index("c")
        base = wid * bpw

        pltpu.sync_copy(idx_hbm.at[pl.ds(base, bpw)], idx_v)

        def start_gather(g, b):
            idx = idx_v.at[pl.ds(pl.multiple_of(g * C, C), C)]
            pltpu.async_copy(table_hbm.at[idx], rows_v[b], gsem.at[b])

        def wait_gather(g, b):
            idx = idx_v.at[pl.ds(pl.multiple_of(g * C, C), C)]
            pltpu.make_async_copy(table_hbm.at[idx], rows_v[b],
                                  gsem.at[b]).wait()

        def start_out(g, b):
            pltpu.async_copy(rows_v[b],
                             out_hbm.at[pl.ds(base + g * C, C)], osem.at[b])

        def wait_out(g, b):
            pltpu.make_async_copy(rows_v[b],
                                  out_hbm.at[pl.ds(base + g * C, C)],
                                  osem.at[b]).wait()

        for b in range(nbuf):
            start_gather(b, b)

        def round_body(i, carry):
            for b in range(nbuf):
                g = i * nbuf + b
                wait_gather(g, b)
                start_out(g, b)
                wait_out(g, b)

                @pl.when(i < rounds - 1)
                def _():
                    start_gather(g + nbuf, b)
            return carry

        lax.fori_loop(0, rounds, round_body, 0)

    return emb


def kernel(x, embeddings):
    bsz, hist = x.shape
    V, D = embeddings.shape
    B = bsz * hist
    xf = x.reshape(B).astype(jnp.int32)
    out = _build(B, V, D, 4, 400)(xf, embeddings)
    return out.reshape(bsz, hist, D)
